# baseline (speedup 1.0000x reference)
import functools

import jax
import jax.numpy as jnp
from jax import lax
from jax.experimental import pallas as pl
from jax.experimental.pallas import tpu as pltpu
from jax.experimental.pallas import tpu_sc as plsc

N = 10000
D = 128
E = 320000

NC = 2
NS = 16
NW = NC * NS

C = 64
NR = 4
NI = 8
EPT = 10240
EP = NW * EPT
CH = EPT // C
NPAD = 10240
ROWS_PER_TILE = NPAD // NS
LANES = 16
SUB = D // LANES


def _matmul_body(x_ref, w_ref, o_ref):
    o_ref[...] = jnp.dot(x_ref[...], w_ref[...], preferred_element_type=jnp.float32)


def _combine_body(p_ref, o_ref):
    o_ref[...] = jnp.maximum(p_ref[0] + p_ref[1], 0.0)


def _sc_spmm_body(packed_hbm, before_hbm, zeros_hbm, out_hbm,
                  idx_v, rows_v, acc, isems, gsems, ssems):
    c = lax.axis_index("c")
    s = lax.axis_index("s")
    wid = s * NC + c
    zbase = pl.multiple_of(s * ROWS_PER_TILE, ROWS_PER_TILE)

    pltpu.sync_copy(zeros_hbm, acc.at[pl.ds(zbase, ROWS_PER_TILE)])

    def issue_idx(g, islot):
        pltpu.async_copy(packed_hbm.at[wid, g], idx_v.at[islot],
                         isems.at[islot])

    def wait_idx(g, islot):
        pltpu.make_async_copy(packed_hbm.at[wid, g], idx_v.at[islot],
                              isems.at[islot]).wait()

    def issue_gather(islot, slot):
        pltpu.async_copy(before_hbm.at[idx_v.at[islot, 0]], rows_v.at[slot],
                         gsems.at[slot])

    def wait_gather(islot, slot):
        pltpu.make_async_copy(before_hbm.at[idx_v.at[islot, 0]],
                              rows_v.at[slot], gsems.at[slot]).wait()

    def scale_chunk(islot, slot):
        def group_body(b, carry2):
            vec = jax.lax.bitcast_convert_type(
                idx_v[islot, 2, pl.ds(b * LANES, LANES)], jnp.float32)
            for k in range(LANES):
                sval = vec[k]
                e = b * LANES + k
                for j in range(SUB):
                    sl = pl.ds(j * LANES, LANES)
                    rows_v[slot, e, sl] = rows_v[slot, e, sl] * sval
            return carry2

        lax.fori_loop(0, C // LANES, group_body, 0)

    def scatter_chunk(islot, slot):
        pltpu.async_copy(rows_v.at[slot], acc.at[idx_v.at[islot, 1]],
                         ssems.at[slot], add=True).wait()

    for k in range(NR):
        issue_idx(k, k)
    plsc.subcore_barrier()
    for k in range(NR - 1):
        wait_idx(k, k)
        issue_gather(k, k)

    def chunk_step(gi, carry):
        g = gi * NI
        for u in range(NI):
            gc = g + u
            islot = u
            islot3 = (u + 3) % NI
            islot4 = (u + 4) % NI
            slot = u % NR
            slot3 = (u + 3) % NR

            @pl.when(gc + 4 < CH)
            def _():
                issue_idx(gc + 4, islot4)

            @pl.when(gc + 3 < CH)
            def _():
                wait_idx(gc + 3, islot3)
                issue_gather(islot3, slot3)
            wait_gather(islot, slot)
            scale_chunk(islot, slot)
            scatter_chunk(islot, slot)
        return carry

    lax.fori_loop(0, CH // NI, chunk_step, 0)
    plsc.subcore_barrier()

    obase = pl.multiple_of(c * NPAD + s * ROWS_PER_TILE, ROWS_PER_TILE)
    pltpu.sync_copy(acc.at[pl.ds(zbase, ROWS_PER_TILE)],
                    out_hbm.at[pl.ds(obase, ROWS_PER_TILE)])


def kernel(inputs, edge_index, adj_values, W):
    before = pl.pallas_call(
        _matmul_body,
        grid=(10,),
        in_specs=[
            pl.BlockSpec((N // 10, D), lambda i: (i, 0)),
            pl.BlockSpec((D, D), lambda i: (0, 0)),
        ],
        out_specs=pl.BlockSpec((N // 10, D), lambda i: (i, 0)),
        out_shape=jax.ShapeDtypeStruct((N, D), jnp.float32),
    )(inputs, W)

    pad = EP - E
    col = jnp.concatenate([edge_index[1], jnp.zeros((pad,), jnp.int32)])
    row = jnp.concatenate([edge_index[0], jnp.zeros((pad,), jnp.int32)])
    adj = jnp.concatenate([adj_values, jnp.zeros((pad,), jnp.float32)])
    adj_bits = jax.lax.bitcast_convert_type(adj, jnp.int32)
    packed = jnp.stack(
        [col.reshape(NW, CH, C), row.reshape(NW, CH, C),
         adj_bits.reshape(NW, CH, C)], axis=2)
    zeros_tile = jnp.zeros((ROWS_PER_TILE, D), jnp.float32)

    mesh = plsc.VectorSubcoreMesh(core_axis_name="c", subcore_axis_name="s")
    sc_spmm = functools.partial(
        pl.kernel,
        out_type=jax.ShapeDtypeStruct((NC * NPAD, D), jnp.float32),
        mesh=mesh,
        scratch_types=[
            pltpu.VMEM((NI, 3, C), jnp.int32),
            pltpu.VMEM((NR, C, D), jnp.float32),
            pltpu.VMEM_SHARED((NPAD, D), jnp.float32),
            pltpu.SemaphoreType.DMA((NI,)),
            pltpu.SemaphoreType.DMA((NR,)),
            pltpu.SemaphoreType.DMA((NR,)),
        ],
    )(_sc_spmm_body)
    partials = sc_spmm(packed, before, zeros_tile)
    partials = partials.reshape(NC, NPAD, D)

    out = pl.pallas_call(
        _combine_body,
        grid=(10,),
        in_specs=[pl.BlockSpec((NC, N // 10, D), lambda i: (0, i, 0))],
        out_specs=pl.BlockSpec((N // 10, D), lambda i: (i, 0)),
        out_shape=jax.ShapeDtypeStruct((N, D), jnp.float32),
    )(partials)
    return out


# speedup vs baseline: 1.1784x; 1.1784x over previous
import functools

import jax
import jax.numpy as jnp
from jax import lax
from jax.experimental import pallas as pl
from jax.experimental.pallas import tpu as pltpu
from jax.experimental.pallas import tpu_sc as plsc

N = 10000
D = 128
E = 320000

NC = 2
NS = 16
NW = NC * NS

C = 128
NR = 2
NI = 4
EPT = 10240
EP = NW * EPT
CH = EPT // C
NPAD = 10112
RPT = NPAD // NS
DW = D // 2
LANES = 16
GRP = D // 32


def _matmul_body(x_ref, w_ref, o_ref):
    b = jnp.dot(x_ref[...], w_ref[...], preferred_element_type=jnp.float32)
    b4 = b.reshape(b.shape[0], GRP, 2, LANES).astype(jnp.bfloat16)
    bits = jax.lax.bitcast_convert_type(b4, jnp.uint16).astype(jnp.uint32)
    word = bits[:, :, 0, :] | (bits[:, :, 1, :] << 16)
    o_ref[...] = jax.lax.bitcast_convert_type(word, jnp.int32).reshape(
        b.shape[0], DW)


def _combine_body(p_ref, o_ref):
    o_ref[...] = jnp.maximum(p_ref[0] + p_ref[1], 0.0)


def _sc_spmm_body(packed_hbm, bpk_hbm, zeros_hbm, out_hbm,
                  idx_v, rows_v, scaled_v, acc, isems, gsems, ssems):
    c = lax.axis_index("c")
    s = lax.axis_index("s")
    wid = s * NC + c
    zbase = pl.multiple_of(s * RPT, RPT)

    pltpu.sync_copy(zeros_hbm, acc.at[pl.ds(zbase, RPT)])

    def issue_idx(g, islot):
        pltpu.async_copy(packed_hbm.at[wid, g], idx_v.at[islot],
                         isems.at[islot])

    def wait_idx(g, islot):
        pltpu.make_async_copy(packed_hbm.at[wid, g], idx_v.at[islot],
                              isems.at[islot]).wait()

    def issue_gather(islot, slot):
        pltpu.async_copy(bpk_hbm.at[idx_v.at[islot, 0]], rows_v.at[slot],
                         gsems.at[slot])

    def wait_gather(islot, slot):
        pltpu.make_async_copy(bpk_hbm.at[idx_v.at[islot, 0]],
                              rows_v.at[slot], gsems.at[slot]).wait()

    def scale_chunk(islot, slot):
        def group_body(b, carry2):
            vec = jax.lax.bitcast_convert_type(
                idx_v[islot, 2, pl.ds(b * LANES, LANES)], jnp.float32)
            for k in range(LANES):
                sval = vec[k]
                e = b * LANES + k
                for g in range(GRP):
                    w = rows_v[slot, e, pl.ds(g * LANES, LANES)]
                    f_lo = jax.lax.bitcast_convert_type(
                        w << 16, jnp.float32)
                    f_hi = jax.lax.bitcast_convert_type(
                        w & jnp.int32(-65536), jnp.float32)
                    scaled_v[e, pl.ds(g * 32, LANES)] = f_lo * sval
                    scaled_v[e, pl.ds(g * 32 + LANES, LANES)] = f_hi * sval
            return carry2

        lax.fori_loop(0, C // LANES, group_body, 0)

    def scatter_chunk(islot):
        pltpu.async_copy(scaled_v, acc.at[idx_v.at[islot, 1]],
                         ssems.at[0], add=True).wait()

    issue_idx(0, 0)
    issue_idx(1, 1)
    plsc.subcore_barrier()
    wait_idx(0, 0)
    issue_gather(0, 0)

    def chunk_step(gi, carry):
        g = gi * NI
        for u in range(NI):
            gc = g + u
            islot = u
            islot1 = (u + 1) % NI
            islot2 = (u + 2) % NI
            slot = u % NR
            slot1 = (u + 1) % NR

            @pl.when(gc + 2 < CH)
            def _():
                issue_idx(gc + 2, islot2)

            @pl.when(gc + 1 < CH)
            def _():
                wait_idx(gc + 1, islot1)
                issue_gather(islot1, slot1)
            wait_gather(islot, slot)
            scale_chunk(islot, slot)
            scatter_chunk(islot)
        return carry

    lax.fori_loop(0, CH // NI, chunk_step, 0)
    plsc.subcore_barrier()

    obase = pl.multiple_of(c * NPAD + zbase, RPT)
    pltpu.sync_copy(acc.at[pl.ds(zbase, RPT)],
                    out_hbm.at[pl.ds(obase, RPT)])


def kernel(inputs, edge_index, adj_values, W):
    bpk = pl.pallas_call(
        _matmul_body,
        grid=(10,),
        in_specs=[
            pl.BlockSpec((N // 10, D), lambda i: (i, 0)),
            pl.BlockSpec((D, D), lambda i: (0, 0)),
        ],
        out_specs=pl.BlockSpec((N // 10, DW), lambda i: (i, 0)),
        out_shape=jax.ShapeDtypeStruct((N, DW), jnp.int32),
    )(inputs, W)

    pad = EP - E
    col = jnp.concatenate([edge_index[1], jnp.zeros((pad,), jnp.int32)])
    row = jnp.concatenate([edge_index[0], jnp.zeros((pad,), jnp.int32)])
    adj = jnp.concatenate([adj_values, jnp.zeros((pad,), jnp.float32)])
    adj_bits = jax.lax.bitcast_convert_type(adj, jnp.int32)
    packed = jnp.stack(
        [col.reshape(NW, CH, C), row.reshape(NW, CH, C),
         adj_bits.reshape(NW, CH, C)], axis=2)
    zeros_tile = jnp.zeros((RPT, D), jnp.float32)

    mesh = plsc.VectorSubcoreMesh(core_axis_name="c", subcore_axis_name="s")
    sc_spmm = functools.partial(
        pl.kernel,
        out_type=jax.ShapeDtypeStruct((NC * NPAD, D), jnp.float32),
        mesh=mesh,
        compiler_params=pltpu.CompilerParams(use_tc_tiling_on_sc=False),
        scratch_types=[
            pltpu.VMEM((NI, 3, C), jnp.int32),
            pltpu.VMEM((NR, C, DW), jnp.int32),
            pltpu.VMEM((C, D), jnp.float32),
            pltpu.VMEM_SHARED((NPAD, D), jnp.float32),
            pltpu.SemaphoreType.DMA((NI,)),
            pltpu.SemaphoreType.DMA((NR,)),
            pltpu.SemaphoreType.DMA((1,)),
        ],
    )(_sc_spmm_body)
    partials = sc_spmm(packed, bpk, zeros_tile)
    partials = partials.reshape(NC, NPAD, D)

    out = pl.pallas_call(
        _combine_body,
        grid=(8,),
        in_specs=[pl.BlockSpec((NC, NPAD // 8, D), lambda i: (0, i, 0))],
        out_specs=pl.BlockSpec((NPAD // 8, D), lambda i: (i, 0)),
        out_shape=jax.ShapeDtypeStruct((NPAD, D), jnp.float32),
    )(partials)
    return out[:N]
